# baseline (speedup 1.0000x reference)
import functools

import jax
import jax.numpy as jnp
from jax import lax
from jax.experimental import pallas as pl
from jax.experimental.pallas import tpu as pltpu
from jax.experimental.pallas import tpu_sc as plsc

NC = 2
NS = 16
L = 16
NW = NC * NS

C = 32


def _emb_body(S, D, TPW, tok_hbm, table_hbm, pos_hbm, out_hbm,
              idx_v, rows_v, pos_v, gsem):
    wid = lax.axis_index("s") * NC + lax.axis_index("c")
    base = wid * TPW
    pos_base = lax.rem(base, S)
    pltpu.sync_copy(tok_hbm.at[pl.ds(base, TPW)], idx_v)
    nvec = D // L

    def chunk_body(g, carry):
        off = g * C
        cp = pltpu.async_copy(
            table_hbm.at[idx_v.at[pl.ds(off, C)]], rows_v, gsem)
        pltpu.sync_copy(pos_hbm.at[pl.ds(pos_base + off, C)], pos_v)
        cp.wait()

        def row_body(r, c2):
            for j in range(nvec):
                sl = pl.ds(j * L, L)
                plsc.addupdate(rows_v.at[r, sl], pos_v[r, sl])
            return c2

        lax.fori_loop(0, C, row_body, 0)
        pltpu.sync_copy(rows_v, out_hbm.at[pl.ds(base + off, C)])
        return carry

    lax.fori_loop(0, TPW // C, chunk_body, 0)


def kernel(input_tokens, token_table, pos_table):
    B, S = input_tokens.shape
    _, D = token_table.shape
    N = B * S
    TPW = N // NW

    mesh = plsc.VectorSubcoreMesh(core_axis_name="c", subcore_axis_name="s")
    k = functools.partial(
        pl.kernel,
        mesh=mesh,
        out_type=jax.ShapeDtypeStruct((N, D), jnp.float32),
        scratch_types=[
            pltpu.VMEM((TPW,), jnp.int32),
            pltpu.VMEM((C, D), jnp.float32),
            pltpu.VMEM((C, D), jnp.float32),
            pltpu.SemaphoreType.DMA,
        ],
    )(functools.partial(_emb_body, S, D, TPW))

    tok_flat = input_tokens.reshape(-1).astype(jnp.int32)
    out = k(tok_flat, token_table, pos_table)
    return out.reshape(B, S, D)


# speedup vs baseline: 2.5796x; 2.5796x over previous
import functools

import jax
import jax.numpy as jnp
from jax import lax
from jax.experimental import pallas as pl
from jax.experimental.pallas import tpu as pltpu
from jax.experimental.pallas import tpu_sc as plsc

NC = 2
NS = 16
L = 16
NW = NC * NS

C = 32


def _emb_body(B, S, D, tok_hbm, table_hbm, pos_hbm, out_hbm,
              idx_v, rows_v, pos_v, g0, g1, s0, s1):
    wid = lax.axis_index("s") * NC + lax.axis_index("c")
    SR = S // NW
    NJ = SR // C
    sbase = wid * SR
    nvec = D // L
    gsem = (g0, g1)
    ssem = (s0, s1)

    for k in range(B):
        pltpu.sync_copy(tok_hbm.at[pl.ds(k * S + sbase, SR)],
                        idx_v.at[pl.ds(k * SR, SR)])

    def issue_gather(j, k, b):
        pltpu.async_copy(
            table_hbm.at[idx_v.at[pl.ds(k * SR + j * C, C)]],
            rows_v.at[b], gsem[b])

    def wait_gather(b):
        pltpu.make_async_copy(
            table_hbm.at[idx_v.at[pl.ds(0, C)]], rows_v.at[b],
            gsem[b]).wait()

    def issue_store(j, k, b):
        pltpu.async_copy(
            rows_v.at[b], out_hbm.at[pl.ds(k * S + sbase + j * C, C)],
            ssem[b])

    def wait_store(b):
        pltpu.make_async_copy(
            rows_v.at[b], out_hbm.at[pl.ds(0, C)], ssem[b]).wait()

    def chunk_body(j, k, first_c, last_c):
        b = k % 2
        nb = 1 - b
        wait_gather(b)
        if k == 0:
            pltpu.sync_copy(pos_hbm.at[pl.ds(sbase + j * C, C)], pos_v)
        if not last_c:
            if not first_c:
                wait_store(nb)
            if k < B - 1:
                issue_gather(j, k + 1, nb)
            else:
                issue_gather(j + 1, 0, nb)

        @plsc.parallel_loop(0, C * nvec, unroll=8)
        def _add(i):
            r = i // nvec
            col = (i % nvec) * L
            plsc.addupdate(rows_v.at[b, r, pl.ds(col, L)],
                           pos_v[r, pl.ds(col, L)])

        issue_store(j, k, b)

    issue_gather(0, 0, 0)
    for k in range(B):
        chunk_body(0, k, first_c=(k == 0), last_c=False)

    def j_body(j, carry):
        for k in range(B):
            chunk_body(j, k, first_c=False, last_c=False)
        return carry

    lax.fori_loop(1, NJ - 1, j_body, 0)

    for k in range(B):
        chunk_body(NJ - 1, k, first_c=False, last_c=(k == B - 1))

    wait_store(0)
    wait_store(1)


def kernel(input_tokens, token_table, pos_table):
    B, S = input_tokens.shape
    _, D = token_table.shape
    N = B * S
    SR = S // NW

    mesh = plsc.VectorSubcoreMesh(core_axis_name="c", subcore_axis_name="s")
    k = functools.partial(
        pl.kernel,
        mesh=mesh,
        out_type=jax.ShapeDtypeStruct((N, D), jnp.float32),
        scratch_types=[
            pltpu.VMEM((B * SR,), jnp.int32),
            pltpu.VMEM((2, C, D), jnp.float32),
            pltpu.VMEM((C, D), jnp.float32),
            pltpu.SemaphoreType.DMA,
            pltpu.SemaphoreType.DMA,
            pltpu.SemaphoreType.DMA,
            pltpu.SemaphoreType.DMA,
        ],
    )(functools.partial(_emb_body, B, S, D))

    tok_flat = input_tokens.reshape(-1).astype(jnp.int32)
    out = k(tok_flat, token_table, pos_table)
    return out.reshape(B, S, D)


# speedup vs baseline: 2.9791x; 1.1549x over previous
import functools

import jax
import jax.numpy as jnp
from jax import lax
from jax.experimental import pallas as pl
from jax.experimental.pallas import tpu as pltpu
from jax.experimental.pallas import tpu_sc as plsc

NC = 2
NS = 16
L = 16
NW = NC * NS

C = 8
NG = 3


def _emb_body(B, S, D, tok_hbm, table_hbm, pos_hbm, out_hbm,
              idx_v, rows_v, pos_v, g0, g1, g2, s0, s1, s2, psem):
    wid = lax.axis_index("s") * NC + lax.axis_index("c")
    SR = S // NW
    NJ = SR // C
    sbase = wid * SR
    nvec = D // L
    gsem = (g0, g1, g2)
    ssem = (s0, s1, s2)

    def issue_pos(j):
        pltpu.async_copy(pos_hbm.at[pl.ds(sbase + j * C, C)], pos_v, psem)

    def wait_pos():
        pltpu.make_async_copy(pos_hbm.at[pl.ds(0, C)], pos_v, psem).wait()

    def issue_gathers(j, g):
        for k in range(B):
            pltpu.async_copy(
                table_hbm.at[idx_v.at[pl.ds(k * SR + j * C, C)]],
                rows_v.at[g, k], gsem[g])

    def wait_gathers(g):
        for k in range(B):
            pltpu.make_async_copy(
                table_hbm.at[idx_v.at[pl.ds(0, C)]], rows_v.at[g, k],
                gsem[g]).wait()

    def issue_stores(j, g):
        for k in range(B):
            pltpu.async_copy(
                rows_v.at[g, k],
                out_hbm.at[pl.ds(k * S + sbase + j * C, C)], ssem[g])

    def wait_stores(g):
        for k in range(B):
            pltpu.make_async_copy(
                rows_v.at[g, k], out_hbm.at[pl.ds(0, C)], ssem[g]).wait()

    def body(j, g, store_wait=True, prefetch=True):
        gn = (g + 1) % NG
        wait_gathers(g)
        if prefetch:
            if store_wait:
                wait_stores(gn)
            issue_gathers(j + 1, gn)
        wait_pos()

        @plsc.parallel_loop(0, C * nvec, unroll=4)
        def _add(i):
            r = i // nvec
            col = (i % nvec) * L
            v = pos_v[r, pl.ds(col, L)]
            for k in range(B):
                plsc.addupdate(rows_v.at[g, k, r, pl.ds(col, L)], v)

        if prefetch:
            issue_pos(j + 1)
        issue_stores(j, g)

    issue_pos(0)
    for k in range(B):
        pltpu.sync_copy(tok_hbm.at[pl.ds(k * S + sbase, SR)],
                        idx_v.at[pl.ds(k * SR, SR)])

    issue_gathers(0, 0)
    body(0, 0, store_wait=False)
    body(1, 1, store_wait=False)
    body(2, 2)
    body(3, 0)

    def super_body(t, carry):
        j = 4 + 3 * t
        body(j, 1)
        body(j + 1, 2)
        body(j + 2, 0)
        return carry

    lax.fori_loop(0, (NJ - 5) // NG, super_body, 0)

    body(NJ - 1, (NJ - 1) % NG, prefetch=False)
    for g in range(NG):
        wait_stores(g)


def kernel(input_tokens, token_table, pos_table):
    B, S = input_tokens.shape
    _, D = token_table.shape
    N = B * S
    SR = S // NW

    mesh = plsc.VectorSubcoreMesh(core_axis_name="c", subcore_axis_name="s")
    k = functools.partial(
        pl.kernel,
        mesh=mesh,
        out_type=jax.ShapeDtypeStruct((N, D), jnp.float32),
        scratch_types=[
            pltpu.VMEM((B * SR,), jnp.int32),
            pltpu.VMEM((NG, B, C, D), jnp.float32),
            pltpu.VMEM((C, D), jnp.float32),
            pltpu.SemaphoreType.DMA,
            pltpu.SemaphoreType.DMA,
            pltpu.SemaphoreType.DMA,
            pltpu.SemaphoreType.DMA,
            pltpu.SemaphoreType.DMA,
            pltpu.SemaphoreType.DMA,
            pltpu.SemaphoreType.DMA,
        ],
    )(functools.partial(_emb_body, B, S, D))

    tok_flat = input_tokens.reshape(-1).astype(jnp.int32)
    out = k(tok_flat, token_table, pos_table)
    return out.reshape(B, S, D)
